# Initial kernel scaffold; baseline (speedup 1.0000x reference)
import jax
import jax.numpy as jnp
from jax.experimental import pallas as pl


def kernel(x, edge_index, batch, W1, b1, W2, b2, W3, b3):
    raise NotImplementedError("write your pallas kernel here")


# speedup vs baseline: 13.4177x; 13.4177x over previous
import functools

import jax
import jax.numpy as jnp
from jax import lax
from jax.experimental import pallas as pl
from jax.experimental.pallas import tpu as pltpu
from jax.experimental.pallas import tpu_sc as plsc

N = 10000
E = 320000
D = 128
G = 16

NC = 2
NS = 16
NW = NC * NS
EPT = E // NW
CH = 80
NCH = EPT // CH
NP = 10240
RPS = NP // NS

BLK = 1000
NBLK = N // BLK

_mesh = plsc.VectorSubcoreMesh(core_axis_name="c", subcore_axis_name="s")


@functools.partial(
    pl.kernel,
    out_type=jax.ShapeDtypeStruct((NC, NP, D), jnp.float32),
    mesh=_mesh,
    scratch_types=[
        pltpu.VMEM((NCH, CH), jnp.int32),
        pltpu.VMEM((NCH, CH), jnp.int32),
        pltpu.VMEM((CH, D), jnp.float32),
        pltpu.VMEM_SHARED((NP, D), jnp.float32),
        pltpu.SemaphoreType.DMA,
    ],
)
def _edge_scatter(g_hbm, src_hbm, dst_hbm, zero_hbm, out_hbm,
                  src_v, dst_v, rows_v, acc_sh, sem):
    cid = lax.axis_index("c")
    sid = lax.axis_index("s")
    wid = sid * NC + cid
    rows0 = sid * RPS
    pltpu.sync_copy(zero_hbm.at[pl.ds(rows0, RPS)], acc_sh.at[pl.ds(rows0, RPS)])
    pltpu.sync_copy(src_hbm.at[wid], src_v)
    pltpu.sync_copy(dst_hbm.at[wid], dst_v)
    plsc.subcore_barrier()

    def body(c, carry):
        pltpu.async_copy(g_hbm.at[src_v.at[c]], rows_v, sem).wait()
        pltpu.sync_copy(rows_v, acc_sh.at[dst_v.at[c]], add=True)
        return carry

    lax.fori_loop(0, NCH, body, 0)
    plsc.subcore_barrier()
    pltpu.sync_copy(acc_sh.at[pl.ds(rows0, RPS)],
                    out_hbm.at[cid, pl.ds(rows0, RPS)])


def _pre_body(dp_ref, x_ref, w_ref, g_ref, dinv_ref):
    deg = dp_ref[0, :, :1] + dp_ref[1, :, :1] + 1.0
    dv = lax.rsqrt(deg)
    dinv_ref[...] = dv
    g_ref[...] = jnp.dot(x_ref[...] * dv, w_ref[...],
                         preferred_element_type=jnp.float32)


def _pre(degp, x, w):
    return pl.pallas_call(
        _pre_body,
        grid=(NBLK,),
        in_specs=[
            pl.BlockSpec((NC, BLK, D), lambda i: (0, i, 0)),
            pl.BlockSpec((BLK, D), lambda i: (i, 0)),
            pl.BlockSpec((D, D), lambda i: (0, 0)),
        ],
        out_specs=[
            pl.BlockSpec((BLK, D), lambda i: (i, 0)),
            pl.BlockSpec((BLK, 1), lambda i: (i, 0)),
        ],
        out_shape=[
            jax.ShapeDtypeStruct((N, D), jnp.float32),
            jax.ShapeDtypeStruct((N, 1), jnp.float32),
        ],
    )(degp, x, w)


def _mid_body(acc_ref, g_ref, dinv_ref, b_ref, w_ref, out_ref):
    s = acc_ref[0] + acc_ref[1] + g_ref[...]
    p = jnp.maximum(dinv_ref[...] * s + b_ref[...], 0.0)
    out_ref[...] = jnp.dot(dinv_ref[...] * p, w_ref[...],
                           preferred_element_type=jnp.float32)


def _mid(acc, g, dinv, b, w):
    return pl.pallas_call(
        _mid_body,
        grid=(NBLK,),
        in_specs=[
            pl.BlockSpec((NC, BLK, D), lambda i: (0, i, 0)),
            pl.BlockSpec((BLK, D), lambda i: (i, 0)),
            pl.BlockSpec((BLK, 1), lambda i: (i, 0)),
            pl.BlockSpec((1, D), lambda i: (0, 0)),
            pl.BlockSpec((D, D), lambda i: (0, 0)),
        ],
        out_specs=pl.BlockSpec((BLK, D), lambda i: (i, 0)),
        out_shape=jax.ShapeDtypeStruct((N, D), jnp.float32),
    )(acc, g, dinv, b, w)


def _final_body(acc_ref, g_ref, dinv_ref, b_ref, batch_ref, out_ref,
                sums_ref, cnt_ref):
    i = pl.program_id(0)
    s = acc_ref[0] + acc_ref[1] + g_ref[...]
    p = jnp.maximum(dinv_ref[...] * s + b_ref[...], 0.0)
    bt = batch_ref[0, 0, :]
    onehot = (bt[:, None] == lax.broadcasted_iota(jnp.int32, (1, G), 1)
              ).astype(jnp.float32)
    psum = lax.dot_general(onehot, p, (((0,), (0,)), ((), ())),
                           preferred_element_type=jnp.float32)
    pcnt = jnp.sum(onehot, axis=0)[:, None]

    @pl.when(i == 0)
    def _():
        sums_ref[...] = psum
        cnt_ref[...] = pcnt

    @pl.when(i > 0)
    def _():
        sums_ref[...] += psum
        cnt_ref[...] += pcnt

    @pl.when(i == NBLK - 1)
    def _():
        out_ref[...] = sums_ref[...] / jnp.maximum(cnt_ref[...], 1.0)


def _final(acc, g, dinv, b, batch3):
    return pl.pallas_call(
        _final_body,
        grid=(NBLK,),
        in_specs=[
            pl.BlockSpec((NC, BLK, D), lambda i: (0, i, 0)),
            pl.BlockSpec((BLK, D), lambda i: (i, 0)),
            pl.BlockSpec((BLK, 1), lambda i: (i, 0)),
            pl.BlockSpec((1, D), lambda i: (0, 0)),
            pl.BlockSpec((1, 1, BLK), lambda i: (i, 0, 0)),
        ],
        out_specs=pl.BlockSpec((G, D), lambda i: (0, 0)),
        out_shape=jax.ShapeDtypeStruct((G, D), jnp.float32),
        scratch_shapes=[
            pltpu.VMEM((G, D), jnp.float32),
            pltpu.VMEM((G, 1), jnp.float32),
        ],
    )(acc, g, dinv, b, batch3)


def kernel(x, edge_index, batch, W1, b1, W2, b2, W3, b3):
    src = edge_index[0].astype(jnp.int32).reshape(NW, NCH, CH)
    dst = edge_index[1].astype(jnp.int32).reshape(NW, NCH, CH)
    zeros_nd = jnp.zeros((NP, D), jnp.float32)
    ones_nd = jnp.ones((N, D), jnp.float32)
    batch3 = batch.astype(jnp.int32).reshape(NBLK, 1, BLK)

    degp = _edge_scatter(ones_nd, dst, dst, zeros_nd)
    g1, dinv = _pre(degp, x, W1)
    acc1 = _edge_scatter(g1, src, dst, zeros_nd)
    g2 = _mid(acc1, g1, dinv, b1.reshape(1, D), W2)
    acc2 = _edge_scatter(g2, src, dst, zeros_nd)
    g3 = _mid(acc2, g2, dinv, b2.reshape(1, D), W3)
    acc3 = _edge_scatter(g3, src, dst, zeros_nd)
    return _final(acc3, g3, dinv, b3.reshape(1, D), batch3)


# speedup vs baseline: 19.5484x; 1.4569x over previous
import functools

import jax
import jax.numpy as jnp
from jax import lax
from jax.experimental import pallas as pl
from jax.experimental.pallas import tpu as pltpu
from jax.experimental.pallas import tpu_sc as plsc

N = 10000
E = 320000
D = 128
G = 16

NC = 2
NS = 16
NW = NC * NS
EPT = E // NW
CH = 100
NCH = EPT // CH
NBUF = 2
NP = 10240
RPS = NP // NS

BLK = 1000
NBLK = N // BLK

_mesh = plsc.VectorSubcoreMesh(core_axis_name="c", subcore_axis_name="s")


@functools.partial(
    pl.kernel,
    out_type=jax.ShapeDtypeStruct((NC, NP, D), jnp.float32),
    mesh=_mesh,
    scratch_types=(
        [pltpu.VMEM((2, CH), jnp.int32)] * NBUF
        + [pltpu.VMEM((CH, D), jnp.float32)] * NBUF
        + [pltpu.VMEM_SHARED((NP, D), jnp.float32)]
        + [pltpu.SemaphoreType.DMA] * (2 * NBUF)
    ),
)
def _edge_scatter(g_hbm, eidx_hbm, zero_hbm, out_hbm, *rest):
    idx = rest[:NBUF]
    rows = rest[NBUF:2 * NBUF]
    acc_sh = rest[2 * NBUF]
    sem_i = rest[2 * NBUF + 1:2 * NBUF + 1 + NBUF]
    sem_g = rest[2 * NBUF + 1 + NBUF:]
    cid = lax.axis_index("c")
    sid = lax.axis_index("s")
    wid = sid * NC + cid
    rows0 = sid * RPS
    pltpu.sync_copy(zero_hbm.at[pl.ds(rows0, RPS)], acc_sh.at[pl.ds(rows0, RPS)])

    for b in range(NBUF):
        pltpu.sync_copy(eidx_hbm.at[wid, b], idx[b])
        pltpu.async_copy(g_hbm.at[idx[b].at[0]], rows[b], sem_g[b])
    plsc.subcore_barrier()

    def body(grp, carry):
        base = grp * NBUF
        for b in range(NBUF):
            c = base + b
            pltpu.make_async_copy(g_hbm.at[idx[b].at[0]], rows[b],
                                  sem_g[b]).wait()
            pltpu.sync_copy(rows[b], acc_sh.at[idx[b].at[1]], add=True)
            nxt = c + NBUF

            @pl.when(nxt < NCH)
            def _():
                pltpu.sync_copy(eidx_hbm.at[wid, nxt], idx[b])
                pltpu.async_copy(g_hbm.at[idx[b].at[0]], rows[b], sem_g[b])

        return carry

    lax.fori_loop(0, NCH // NBUF, body, 0)
    plsc.subcore_barrier()
    pltpu.sync_copy(acc_sh.at[pl.ds(rows0, RPS)],
                    out_hbm.at[cid, pl.ds(rows0, RPS)])


def _pre_body(dp_ref, x_ref, w_ref, g_ref, dinv_ref):
    deg = dp_ref[0, :, :1] + dp_ref[1, :, :1] + 1.0
    dv = lax.rsqrt(deg)
    dinv_ref[...] = dv
    g_ref[...] = jnp.dot(x_ref[...] * dv, w_ref[...],
                         preferred_element_type=jnp.float32)


def _pre(degp, x, w):
    return pl.pallas_call(
        _pre_body,
        grid=(NBLK,),
        in_specs=[
            pl.BlockSpec((NC, BLK, D), lambda i: (0, i, 0)),
            pl.BlockSpec((BLK, D), lambda i: (i, 0)),
            pl.BlockSpec((D, D), lambda i: (0, 0)),
        ],
        out_specs=[
            pl.BlockSpec((BLK, D), lambda i: (i, 0)),
            pl.BlockSpec((BLK, 1), lambda i: (i, 0)),
        ],
        out_shape=[
            jax.ShapeDtypeStruct((N, D), jnp.float32),
            jax.ShapeDtypeStruct((N, 1), jnp.float32),
        ],
    )(degp, x, w)


def _mid_body(acc_ref, g_ref, dinv_ref, b_ref, w_ref, out_ref):
    s = acc_ref[0] + acc_ref[1] + g_ref[...]
    p = jnp.maximum(dinv_ref[...] * s + b_ref[...], 0.0)
    out_ref[...] = jnp.dot(dinv_ref[...] * p, w_ref[...],
                           preferred_element_type=jnp.float32)


def _mid(acc, g, dinv, b, w):
    return pl.pallas_call(
        _mid_body,
        grid=(NBLK,),
        in_specs=[
            pl.BlockSpec((NC, BLK, D), lambda i: (0, i, 0)),
            pl.BlockSpec((BLK, D), lambda i: (i, 0)),
            pl.BlockSpec((BLK, 1), lambda i: (i, 0)),
            pl.BlockSpec((1, D), lambda i: (0, 0)),
            pl.BlockSpec((D, D), lambda i: (0, 0)),
        ],
        out_specs=pl.BlockSpec((BLK, D), lambda i: (i, 0)),
        out_shape=jax.ShapeDtypeStruct((N, D), jnp.float32),
    )(acc, g, dinv, b, w)


def _final_body(acc_ref, g_ref, dinv_ref, b_ref, batch_ref, out_ref,
                sums_ref, cnt_ref):
    i = pl.program_id(0)
    s = acc_ref[0] + acc_ref[1] + g_ref[...]
    p = jnp.maximum(dinv_ref[...] * s + b_ref[...], 0.0)
    bt = batch_ref[0, 0, :]
    onehot = (bt[:, None] == lax.broadcasted_iota(jnp.int32, (1, G), 1)
              ).astype(jnp.float32)
    psum = lax.dot_general(onehot, p, (((0,), (0,)), ((), ())),
                           preferred_element_type=jnp.float32)
    pcnt = jnp.sum(onehot, axis=0)[:, None]

    @pl.when(i == 0)
    def _():
        sums_ref[...] = psum
        cnt_ref[...] = pcnt

    @pl.when(i > 0)
    def _():
        sums_ref[...] += psum
        cnt_ref[...] += pcnt

    @pl.when(i == NBLK - 1)
    def _():
        out_ref[...] = sums_ref[...] / jnp.maximum(cnt_ref[...], 1.0)


def _final(acc, g, dinv, b, batch3):
    return pl.pallas_call(
        _final_body,
        grid=(NBLK,),
        in_specs=[
            pl.BlockSpec((NC, BLK, D), lambda i: (0, i, 0)),
            pl.BlockSpec((BLK, D), lambda i: (i, 0)),
            pl.BlockSpec((BLK, 1), lambda i: (i, 0)),
            pl.BlockSpec((1, D), lambda i: (0, 0)),
            pl.BlockSpec((1, 1, BLK), lambda i: (i, 0, 0)),
        ],
        out_specs=pl.BlockSpec((G, D), lambda i: (0, 0)),
        out_shape=jax.ShapeDtypeStruct((G, D), jnp.float32),
        scratch_shapes=[
            pltpu.VMEM((G, D), jnp.float32),
            pltpu.VMEM((G, 1), jnp.float32),
        ],
    )(acc, g, dinv, b, batch3)


def kernel(x, edge_index, batch, W1, b1, W2, b2, W3, b3):
    src = edge_index[0].astype(jnp.int32).reshape(NW, NCH, 1, CH)
    dst = edge_index[1].astype(jnp.int32).reshape(NW, NCH, 1, CH)
    eidx = jnp.concatenate([src, dst], axis=2)
    didx = jnp.concatenate([dst, dst], axis=2)
    zeros_nd = jnp.zeros((NP, D), jnp.float32)
    ones_nd = jnp.ones((N, D), jnp.float32)
    batch3 = batch.astype(jnp.int32).reshape(NBLK, 1, BLK)

    degp = _edge_scatter(ones_nd, didx, zeros_nd)
    g1, dinv = _pre(degp, x, W1)
    acc1 = _edge_scatter(g1, eidx, zeros_nd)
    g2 = _mid(acc1, g1, dinv, b1.reshape(1, D), W2)
    acc2 = _edge_scatter(g2, eidx, zeros_nd)
    g3 = _mid(acc2, g2, dinv, b2.reshape(1, D), W3)
    acc3 = _edge_scatter(g3, eidx, zeros_nd)
    return _final(acc3, g3, dinv, b3.reshape(1, D), batch3)


# speedup vs baseline: 22.7865x; 1.1656x over previous
import functools

import jax
import jax.numpy as jnp
from jax import lax
from jax.experimental import pallas as pl
from jax.experimental.pallas import tpu as pltpu
from jax.experimental.pallas import tpu_sc as plsc

N = 10000
E = 320000
D = 128
G = 16

NC = 2
NS = 16
NW = NC * NS
EPT = E // NW
CH = 100
NCH = EPT // CH
NBUF = 2
NIDX = 4
NP = 10240
RPS = NP // NS

BLK = 1000
NBLK = N // BLK

_mesh = plsc.VectorSubcoreMesh(core_axis_name="c", subcore_axis_name="s")


@functools.partial(
    pl.kernel,
    out_type=jax.ShapeDtypeStruct((NC, NP, D), jnp.float32),
    mesh=_mesh,
    scratch_types=(
        [pltpu.VMEM((2, CH), jnp.int32)] * NIDX
        + [pltpu.VMEM((CH, D), jnp.float32)] * NBUF
        + [pltpu.VMEM_SHARED((NP, D), jnp.float32)]
        + [pltpu.SemaphoreType.DMA] * (NIDX + NBUF)
    ),
)
def _edge_scatter(g_hbm, eidx_hbm, zero_hbm, out_hbm, *rest):
    idx = rest[:NIDX]
    rows = rest[NIDX:NIDX + NBUF]
    acc_sh = rest[NIDX + NBUF]
    sem_i = rest[NIDX + NBUF + 1:NIDX + NBUF + 1 + NIDX]
    sem_g = rest[NIDX + NBUF + 1 + NIDX:]
    cid = lax.axis_index("c")
    sid = lax.axis_index("s")
    wid = sid * NC + cid
    rows0 = sid * RPS
    pltpu.sync_copy(zero_hbm.at[pl.ds(rows0, RPS)], acc_sh.at[pl.ds(rows0, RPS)])

    for k in range(NIDX):
        pltpu.async_copy(eidx_hbm.at[wid, k], idx[k], sem_i[k])
    for b in range(NBUF):
        pltpu.make_async_copy(eidx_hbm.at[wid, b], idx[b], sem_i[b]).wait()
        pltpu.async_copy(g_hbm.at[idx[b].at[0]], rows[b], sem_g[b])
    plsc.subcore_barrier()

    def body(grp, carry):
        base = grp * NIDX
        for j in range(NIDX):
            c = base + j
            b = j % NBUF
            k = j % NIDX
            pltpu.make_async_copy(g_hbm.at[idx[k].at[0]], rows[b],
                                  sem_g[b]).wait()
            pltpu.sync_copy(rows[b], acc_sh.at[idx[k].at[1]], add=True)
            ni = c + NIDX

            @pl.when(ni < NCH)
            def _():
                pltpu.async_copy(eidx_hbm.at[wid, ni], idx[k], sem_i[k])

            ng = c + NBUF

            @pl.when(ng < NCH)
            def _():
                kg = (j + NBUF) % NIDX
                pltpu.make_async_copy(eidx_hbm.at[wid, ng], idx[kg],
                                      sem_i[kg]).wait()
                pltpu.async_copy(g_hbm.at[idx[kg].at[0]], rows[b], sem_g[b])

        return carry

    lax.fori_loop(0, NCH // NIDX, body, 0)
    plsc.subcore_barrier()
    pltpu.sync_copy(acc_sh.at[pl.ds(rows0, RPS)],
                    out_hbm.at[cid, pl.ds(rows0, RPS)])


def _pre_body(dp_ref, x_ref, w_ref, g_ref, dinv_ref):
    deg = dp_ref[0, :, :1] + dp_ref[1, :, :1] + 1.0
    dv = lax.rsqrt(deg)
    dinv_ref[...] = dv
    g_ref[...] = jnp.dot(x_ref[...] * dv, w_ref[...],
                         preferred_element_type=jnp.float32)


def _pre(degp, x, w):
    return pl.pallas_call(
        _pre_body,
        grid=(NBLK,),
        in_specs=[
            pl.BlockSpec((NC, BLK, D), lambda i: (0, i, 0)),
            pl.BlockSpec((BLK, D), lambda i: (i, 0)),
            pl.BlockSpec((D, D), lambda i: (0, 0)),
        ],
        out_specs=[
            pl.BlockSpec((BLK, D), lambda i: (i, 0)),
            pl.BlockSpec((BLK, 1), lambda i: (i, 0)),
        ],
        out_shape=[
            jax.ShapeDtypeStruct((N, D), jnp.float32),
            jax.ShapeDtypeStruct((N, 1), jnp.float32),
        ],
    )(degp, x, w)


def _mid_body(acc_ref, g_ref, dinv_ref, b_ref, w_ref, out_ref):
    s = acc_ref[0] + acc_ref[1] + g_ref[...]
    p = jnp.maximum(dinv_ref[...] * s + b_ref[...], 0.0)
    out_ref[...] = jnp.dot(dinv_ref[...] * p, w_ref[...],
                           preferred_element_type=jnp.float32)


def _mid(acc, g, dinv, b, w):
    return pl.pallas_call(
        _mid_body,
        grid=(NBLK,),
        in_specs=[
            pl.BlockSpec((NC, BLK, D), lambda i: (0, i, 0)),
            pl.BlockSpec((BLK, D), lambda i: (i, 0)),
            pl.BlockSpec((BLK, 1), lambda i: (i, 0)),
            pl.BlockSpec((1, D), lambda i: (0, 0)),
            pl.BlockSpec((D, D), lambda i: (0, 0)),
        ],
        out_specs=pl.BlockSpec((BLK, D), lambda i: (i, 0)),
        out_shape=jax.ShapeDtypeStruct((N, D), jnp.float32),
    )(acc, g, dinv, b, w)


def _final_body(acc_ref, g_ref, dinv_ref, b_ref, batch_ref, out_ref,
                sums_ref, cnt_ref):
    i = pl.program_id(0)
    s = acc_ref[0] + acc_ref[1] + g_ref[...]
    p = jnp.maximum(dinv_ref[...] * s + b_ref[...], 0.0)
    bt = batch_ref[0, 0, :]
    onehot = (bt[:, None] == lax.broadcasted_iota(jnp.int32, (1, G), 1)
              ).astype(jnp.float32)
    psum = lax.dot_general(onehot, p, (((0,), (0,)), ((), ())),
                           preferred_element_type=jnp.float32)
    pcnt = jnp.sum(onehot, axis=0)[:, None]

    @pl.when(i == 0)
    def _():
        sums_ref[...] = psum
        cnt_ref[...] = pcnt

    @pl.when(i > 0)
    def _():
        sums_ref[...] += psum
        cnt_ref[...] += pcnt

    @pl.when(i == NBLK - 1)
    def _():
        out_ref[...] = sums_ref[...] / jnp.maximum(cnt_ref[...], 1.0)


def _final(acc, g, dinv, b, batch3):
    return pl.pallas_call(
        _final_body,
        grid=(NBLK,),
        in_specs=[
            pl.BlockSpec((NC, BLK, D), lambda i: (0, i, 0)),
            pl.BlockSpec((BLK, D), lambda i: (i, 0)),
            pl.BlockSpec((BLK, 1), lambda i: (i, 0)),
            pl.BlockSpec((1, D), lambda i: (0, 0)),
            pl.BlockSpec((1, 1, BLK), lambda i: (i, 0, 0)),
        ],
        out_specs=pl.BlockSpec((G, D), lambda i: (0, 0)),
        out_shape=jax.ShapeDtypeStruct((G, D), jnp.float32),
        scratch_shapes=[
            pltpu.VMEM((G, D), jnp.float32),
            pltpu.VMEM((G, 1), jnp.float32),
        ],
    )(acc, g, dinv, b, batch3)


def kernel(x, edge_index, batch, W1, b1, W2, b2, W3, b3):
    src = edge_index[0].astype(jnp.int32).reshape(NW, NCH, 1, CH)
    dst = edge_index[1].astype(jnp.int32).reshape(NW, NCH, 1, CH)
    eidx = jnp.concatenate([src, dst], axis=2)
    didx = jnp.concatenate([dst, dst], axis=2)
    zeros_nd = jnp.zeros((NP, D), jnp.float32)
    ones_nd = jnp.ones((N, D), jnp.float32)
    batch3 = batch.astype(jnp.int32).reshape(NBLK, 1, BLK)

    degp = _edge_scatter(ones_nd, didx, zeros_nd)
    g1, dinv = _pre(degp, x, W1)
    acc1 = _edge_scatter(g1, eidx, zeros_nd)
    g2 = _mid(acc1, g1, dinv, b1.reshape(1, D), W2)
    acc2 = _edge_scatter(g2, eidx, zeros_nd)
    g3 = _mid(acc2, g2, dinv, b2.reshape(1, D), W3)
    acc3 = _edge_scatter(g3, eidx, zeros_nd)
    return _final(acc3, g3, dinv, b3.reshape(1, D), batch3)


# speedup vs baseline: 23.8332x; 1.0459x over previous
import functools

import jax
import jax.numpy as jnp
from jax import lax
from jax.experimental import pallas as pl
from jax.experimental.pallas import tpu as pltpu
from jax.experimental.pallas import tpu_sc as plsc

N = 10000
E = 320000
D = 128
G = 16

NC = 2
NS = 16
NW = NC * NS
EPT = E // NW
CH = 125
NCH = EPT // CH
NBUF = 2
NIDX = 4
NP = 10240
RPS = NP // NS

BLK = 1000
NBLK = N // BLK

_mesh = plsc.VectorSubcoreMesh(core_axis_name="c", subcore_axis_name="s")


@functools.partial(
    pl.kernel,
    out_type=jax.ShapeDtypeStruct((NC, NP, D), jnp.float32),
    mesh=_mesh,
    scratch_types=(
        [pltpu.VMEM((2, CH), jnp.int32)] * NIDX
        + [pltpu.VMEM((CH, D), jnp.float32)] * NBUF
        + [pltpu.VMEM_SHARED((NP, D), jnp.float32)]
        + [pltpu.SemaphoreType.DMA] * (NIDX + NBUF)
    ),
)
def _edge_scatter(g_hbm, eidx_hbm, zero_hbm, out_hbm, *rest):
    idx = rest[:NIDX]
    rows = rest[NIDX:NIDX + NBUF]
    acc_sh = rest[NIDX + NBUF]
    sem_i = rest[NIDX + NBUF + 1:NIDX + NBUF + 1 + NIDX]
    sem_g = rest[NIDX + NBUF + 1 + NIDX:]
    cid = lax.axis_index("c")
    sid = lax.axis_index("s")
    wid = sid * NC + cid
    rows0 = sid * RPS
    pltpu.sync_copy(zero_hbm.at[pl.ds(rows0, RPS)], acc_sh.at[pl.ds(rows0, RPS)])

    for k in range(NIDX):
        pltpu.async_copy(eidx_hbm.at[wid, k], idx[k], sem_i[k])
    for b in range(NBUF):
        pltpu.make_async_copy(eidx_hbm.at[wid, b], idx[b], sem_i[b]).wait()
        pltpu.async_copy(g_hbm.at[idx[b].at[0]], rows[b], sem_g[b])
    plsc.subcore_barrier()

    def body(grp, carry):
        base = grp * NIDX
        for j in range(NIDX):
            c = base + j
            b = j % NBUF
            k = j % NIDX
            pltpu.make_async_copy(g_hbm.at[idx[k].at[0]], rows[b],
                                  sem_g[b]).wait()
            pltpu.sync_copy(rows[b], acc_sh.at[idx[k].at[1]], add=True)
            ni = c + NIDX

            @pl.when(ni < NCH)
            def _():
                pltpu.async_copy(eidx_hbm.at[wid, ni], idx[k], sem_i[k])

            ng = c + NBUF

            @pl.when(ng < NCH)
            def _():
                kg = (j + NBUF) % NIDX
                pltpu.make_async_copy(eidx_hbm.at[wid, ng], idx[kg],
                                      sem_i[kg]).wait()
                pltpu.async_copy(g_hbm.at[idx[kg].at[0]], rows[b], sem_g[b])

        return carry

    lax.fori_loop(0, NCH // NIDX, body, 0)
    plsc.subcore_barrier()
    pltpu.sync_copy(acc_sh.at[pl.ds(rows0, RPS)],
                    out_hbm.at[cid, pl.ds(rows0, RPS)])


def _pre_body(dp_ref, x_ref, w_ref, g_ref, dinv_ref):
    deg = dp_ref[0, :, :1] + dp_ref[1, :, :1] + 1.0
    dv = lax.rsqrt(deg)
    dinv_ref[...] = dv
    g_ref[...] = jnp.dot(x_ref[...] * dv, w_ref[...],
                         preferred_element_type=jnp.float32)


def _pre(degp, x, w):
    return pl.pallas_call(
        _pre_body,
        grid=(NBLK,),
        in_specs=[
            pl.BlockSpec((NC, BLK, D), lambda i: (0, i, 0)),
            pl.BlockSpec((BLK, D), lambda i: (i, 0)),
            pl.BlockSpec((D, D), lambda i: (0, 0)),
        ],
        out_specs=[
            pl.BlockSpec((BLK, D), lambda i: (i, 0)),
            pl.BlockSpec((BLK, 1), lambda i: (i, 0)),
        ],
        out_shape=[
            jax.ShapeDtypeStruct((N, D), jnp.float32),
            jax.ShapeDtypeStruct((N, 1), jnp.float32),
        ],
    )(degp, x, w)


def _mid_body(acc_ref, g_ref, dinv_ref, b_ref, w_ref, out_ref):
    s = acc_ref[0] + acc_ref[1] + g_ref[...]
    p = jnp.maximum(dinv_ref[...] * s + b_ref[...], 0.0)
    out_ref[...] = jnp.dot(dinv_ref[...] * p, w_ref[...],
                           preferred_element_type=jnp.float32)


def _mid(acc, g, dinv, b, w):
    return pl.pallas_call(
        _mid_body,
        grid=(NBLK,),
        in_specs=[
            pl.BlockSpec((NC, BLK, D), lambda i: (0, i, 0)),
            pl.BlockSpec((BLK, D), lambda i: (i, 0)),
            pl.BlockSpec((BLK, 1), lambda i: (i, 0)),
            pl.BlockSpec((1, D), lambda i: (0, 0)),
            pl.BlockSpec((D, D), lambda i: (0, 0)),
        ],
        out_specs=pl.BlockSpec((BLK, D), lambda i: (i, 0)),
        out_shape=jax.ShapeDtypeStruct((N, D), jnp.float32),
    )(acc, g, dinv, b, w)


def _final_body(acc_ref, g_ref, dinv_ref, b_ref, batch_ref, out_ref,
                sums_ref, cnt_ref):
    i = pl.program_id(0)
    s = acc_ref[0] + acc_ref[1] + g_ref[...]
    p = jnp.maximum(dinv_ref[...] * s + b_ref[...], 0.0)
    bt = batch_ref[0, 0, :]
    onehot = (bt[:, None] == lax.broadcasted_iota(jnp.int32, (1, G), 1)
              ).astype(jnp.float32)
    psum = lax.dot_general(onehot, p, (((0,), (0,)), ((), ())),
                           preferred_element_type=jnp.float32)
    pcnt = jnp.sum(onehot, axis=0)[:, None]

    @pl.when(i == 0)
    def _():
        sums_ref[...] = psum
        cnt_ref[...] = pcnt

    @pl.when(i > 0)
    def _():
        sums_ref[...] += psum
        cnt_ref[...] += pcnt

    @pl.when(i == NBLK - 1)
    def _():
        out_ref[...] = sums_ref[...] / jnp.maximum(cnt_ref[...], 1.0)


def _final(acc, g, dinv, b, batch3):
    return pl.pallas_call(
        _final_body,
        grid=(NBLK,),
        in_specs=[
            pl.BlockSpec((NC, BLK, D), lambda i: (0, i, 0)),
            pl.BlockSpec((BLK, D), lambda i: (i, 0)),
            pl.BlockSpec((BLK, 1), lambda i: (i, 0)),
            pl.BlockSpec((1, D), lambda i: (0, 0)),
            pl.BlockSpec((1, 1, BLK), lambda i: (i, 0, 0)),
        ],
        out_specs=pl.BlockSpec((G, D), lambda i: (0, 0)),
        out_shape=jax.ShapeDtypeStruct((G, D), jnp.float32),
        scratch_shapes=[
            pltpu.VMEM((G, D), jnp.float32),
            pltpu.VMEM((G, 1), jnp.float32),
        ],
    )(acc, g, dinv, b, batch3)


def kernel(x, edge_index, batch, W1, b1, W2, b2, W3, b3):
    src = edge_index[0].astype(jnp.int32).reshape(NW, NCH, 1, CH)
    dst = edge_index[1].astype(jnp.int32).reshape(NW, NCH, 1, CH)
    eidx = jnp.concatenate([src, dst], axis=2)
    didx = jnp.concatenate([dst, dst], axis=2)
    zeros_nd = jnp.zeros((NP, D), jnp.float32)
    ones_nd = jnp.ones((N, D), jnp.float32)
    batch3 = batch.astype(jnp.int32).reshape(NBLK, 1, BLK)

    degp = _edge_scatter(ones_nd, didx, zeros_nd)
    g1, dinv = _pre(degp, x, W1)
    acc1 = _edge_scatter(g1, eidx, zeros_nd)
    g2 = _mid(acc1, g1, dinv, b1.reshape(1, D), W2)
    acc2 = _edge_scatter(g2, eidx, zeros_nd)
    g3 = _mid(acc2, g2, dinv, b2.reshape(1, D), W3)
    acc3 = _edge_scatter(g3, eidx, zeros_nd)
    return _final(acc3, g3, dinv, b3.reshape(1, D), batch3)


# speedup vs baseline: 24.3139x; 1.0202x over previous
import functools

import jax
import jax.numpy as jnp
from jax import lax
from jax.experimental import pallas as pl
from jax.experimental.pallas import tpu as pltpu
from jax.experimental.pallas import tpu_sc as plsc

N = 10000
E = 320000
D = 128
G = 16

NC = 2
NS = 16
NW = NC * NS
EPT = E // NW
CH = 125
NCH = EPT // CH
NBUF = 2
NIDX = 4
NP = 10240
RPS = NP // NS

BLK = 2000
NBLK = N // BLK

_mesh = plsc.VectorSubcoreMesh(core_axis_name="c", subcore_axis_name="s")


@functools.partial(
    pl.kernel,
    out_type=jax.ShapeDtypeStruct((NC, NP, D), jnp.float32),
    mesh=_mesh,
    scratch_types=(
        [pltpu.VMEM((2, CH), jnp.int32)] * NIDX
        + [pltpu.VMEM((CH, D), jnp.float32)] * NBUF
        + [pltpu.VMEM_SHARED((NP, D), jnp.float32)]
        + [pltpu.SemaphoreType.DMA] * (NIDX + NBUF)
    ),
)
def _edge_scatter(g_hbm, eidx_hbm, zero_hbm, out_hbm, *rest):
    idx = rest[:NIDX]
    rows = rest[NIDX:NIDX + NBUF]
    acc_sh = rest[NIDX + NBUF]
    sem_i = rest[NIDX + NBUF + 1:NIDX + NBUF + 1 + NIDX]
    sem_g = rest[NIDX + NBUF + 1 + NIDX:]
    cid = lax.axis_index("c")
    sid = lax.axis_index("s")
    wid = sid * NC + cid
    rows0 = sid * RPS
    pltpu.sync_copy(zero_hbm.at[pl.ds(rows0, RPS)], acc_sh.at[pl.ds(rows0, RPS)])

    for k in range(NIDX):
        pltpu.async_copy(eidx_hbm.at[wid, k], idx[k], sem_i[k])
    for b in range(NBUF):
        pltpu.make_async_copy(eidx_hbm.at[wid, b], idx[b], sem_i[b]).wait()
        pltpu.async_copy(g_hbm.at[idx[b].at[0]], rows[b], sem_g[b])
    plsc.subcore_barrier()

    def body(grp, carry):
        base = grp * NIDX
        for j in range(NIDX):
            c = base + j
            b = j % NBUF
            k = j % NIDX
            pltpu.make_async_copy(g_hbm.at[idx[k].at[0]], rows[b],
                                  sem_g[b]).wait()
            pltpu.sync_copy(rows[b], acc_sh.at[idx[k].at[1]], add=True)
            ni = c + NIDX

            @pl.when(ni < NCH)
            def _():
                pltpu.async_copy(eidx_hbm.at[wid, ni], idx[k], sem_i[k])

            ng = c + NBUF

            @pl.when(ng < NCH)
            def _():
                kg = (j + NBUF) % NIDX
                pltpu.make_async_copy(eidx_hbm.at[wid, ng], idx[kg],
                                      sem_i[kg]).wait()
                pltpu.async_copy(g_hbm.at[idx[kg].at[0]], rows[b], sem_g[b])

        return carry

    lax.fori_loop(0, NCH // NIDX, body, 0)
    plsc.subcore_barrier()
    pltpu.sync_copy(acc_sh.at[pl.ds(rows0, RPS)],
                    out_hbm.at[cid, pl.ds(rows0, RPS)])


def _pre_body(dp_ref, x_ref, w_ref, g_ref, dinv_ref):
    deg = dp_ref[0, :, :1] + dp_ref[1, :, :1] + 1.0
    dv = lax.rsqrt(deg)
    dinv_ref[...] = dv
    g_ref[...] = jnp.dot(x_ref[...] * dv, w_ref[...],
                         preferred_element_type=jnp.float32)


def _pre(degp, x, w):
    return pl.pallas_call(
        _pre_body,
        grid=(NBLK,),
        in_specs=[
            pl.BlockSpec((NC, BLK, D), lambda i: (0, i, 0)),
            pl.BlockSpec((BLK, D), lambda i: (i, 0)),
            pl.BlockSpec((D, D), lambda i: (0, 0)),
        ],
        out_specs=[
            pl.BlockSpec((BLK, D), lambda i: (i, 0)),
            pl.BlockSpec((BLK, 1), lambda i: (i, 0)),
        ],
        out_shape=[
            jax.ShapeDtypeStruct((N, D), jnp.float32),
            jax.ShapeDtypeStruct((N, 1), jnp.float32),
        ],
    )(degp, x, w)


def _mid_body(acc_ref, g_ref, dinv_ref, b_ref, w_ref, out_ref):
    s = acc_ref[0] + acc_ref[1] + g_ref[...]
    p = jnp.maximum(dinv_ref[...] * s + b_ref[...], 0.0)
    out_ref[...] = jnp.dot(dinv_ref[...] * p, w_ref[...],
                           preferred_element_type=jnp.float32)


def _mid(acc, g, dinv, b, w):
    return pl.pallas_call(
        _mid_body,
        grid=(NBLK,),
        in_specs=[
            pl.BlockSpec((NC, BLK, D), lambda i: (0, i, 0)),
            pl.BlockSpec((BLK, D), lambda i: (i, 0)),
            pl.BlockSpec((BLK, 1), lambda i: (i, 0)),
            pl.BlockSpec((1, D), lambda i: (0, 0)),
            pl.BlockSpec((D, D), lambda i: (0, 0)),
        ],
        out_specs=pl.BlockSpec((BLK, D), lambda i: (i, 0)),
        out_shape=jax.ShapeDtypeStruct((N, D), jnp.float32),
    )(acc, g, dinv, b, w)


def _final_body(acc_ref, g_ref, dinv_ref, b_ref, batch_ref, out_ref,
                sums_ref, cnt_ref):
    i = pl.program_id(0)
    s = acc_ref[0] + acc_ref[1] + g_ref[...]
    p = jnp.maximum(dinv_ref[...] * s + b_ref[...], 0.0)
    bt = batch_ref[0, 0, :]
    onehot = (bt[:, None] == lax.broadcasted_iota(jnp.int32, (1, G), 1)
              ).astype(jnp.float32)
    psum = lax.dot_general(onehot, p, (((0,), (0,)), ((), ())),
                           preferred_element_type=jnp.float32)
    pcnt = jnp.sum(onehot, axis=0)[:, None]

    @pl.when(i == 0)
    def _():
        sums_ref[...] = psum
        cnt_ref[...] = pcnt

    @pl.when(i > 0)
    def _():
        sums_ref[...] += psum
        cnt_ref[...] += pcnt

    @pl.when(i == NBLK - 1)
    def _():
        out_ref[...] = sums_ref[...] / jnp.maximum(cnt_ref[...], 1.0)


def _final(acc, g, dinv, b, batch3):
    return pl.pallas_call(
        _final_body,
        grid=(NBLK,),
        in_specs=[
            pl.BlockSpec((NC, BLK, D), lambda i: (0, i, 0)),
            pl.BlockSpec((BLK, D), lambda i: (i, 0)),
            pl.BlockSpec((BLK, 1), lambda i: (i, 0)),
            pl.BlockSpec((1, D), lambda i: (0, 0)),
            pl.BlockSpec((1, 1, BLK), lambda i: (i, 0, 0)),
        ],
        out_specs=pl.BlockSpec((G, D), lambda i: (0, 0)),
        out_shape=jax.ShapeDtypeStruct((G, D), jnp.float32),
        scratch_shapes=[
            pltpu.VMEM((G, D), jnp.float32),
            pltpu.VMEM((G, 1), jnp.float32),
        ],
    )(acc, g, dinv, b, batch3)


def kernel(x, edge_index, batch, W1, b1, W2, b2, W3, b3):
    src = edge_index[0].astype(jnp.int32).reshape(NW, NCH, 1, CH)
    dst = edge_index[1].astype(jnp.int32).reshape(NW, NCH, 1, CH)
    eidx = jnp.concatenate([src, dst], axis=2)
    zeros_nd = jnp.zeros((NP, D), jnp.float32)
    ones_nd = jnp.ones((N, D), jnp.float32)
    batch3 = batch.astype(jnp.int32).reshape(NBLK, 1, BLK)

    degp = _edge_scatter(ones_nd, eidx, zeros_nd)
    g1, dinv = _pre(degp, x, W1)
    acc1 = _edge_scatter(g1, eidx, zeros_nd)
    g2 = _mid(acc1, g1, dinv, b1.reshape(1, D), W2)
    acc2 = _edge_scatter(g2, eidx, zeros_nd)
    g3 = _mid(acc2, g2, dinv, b2.reshape(1, D), W3)
    acc3 = _edge_scatter(g3, eidx, zeros_nd)
    return _final(acc3, g3, dinv, b3.reshape(1, D), batch3)
